# baseline (speedup 1.0000x reference)
import functools

import jax
import jax.numpy as jnp
from jax import lax
from jax.experimental import pallas as pl
from jax.experimental.pallas import tpu as pltpu
from jax.experimental.pallas import tpu_sc as plsc

N = 10000
E = 320000
D = 128
DE = 16

NC = 2
NS = 16
L = 16
NW = NC * NS
CH = 64

BM_E = 8000
BM_N = 2000


def _mm_kernel(x_ref, w_ref, b_ref, o_ref):
    o_ref[...] = (
        jnp.dot(x_ref[...], w_ref[...], preferred_element_type=jnp.float32)
        + b_ref[...]
    )


def _mm(x, w, b, bm):
    m, k = x.shape
    dout = w.shape[1]
    return pl.pallas_call(
        _mm_kernel,
        grid=(m // bm,),
        in_specs=[
            pl.BlockSpec((bm, k), lambda i: (i, 0)),
            pl.BlockSpec((k, dout), lambda i: (0, 0)),
            pl.BlockSpec((1, dout), lambda i: (0, 0)),
        ],
        out_specs=pl.BlockSpec((bm, dout), lambda i: (i, 0)),
        out_shape=jax.ShapeDtypeStruct((m, dout), jnp.float32),
    )(x, w, b.reshape(1, -1))


E8 = E // 8
BME8 = BM_E // 8


def _edge_a_kernel(ea_ref, w1_ref, b1_ref, w2_ref, b2_ref, o1_ref, o2_ref):
    ea = ea_ref[...]
    o1_ref[...] = (
        jnp.dot(ea, w1_ref[...], preferred_element_type=jnp.float32) + b1_ref[...]
    )
    o2_ref[...] = (
        jnp.dot(ea, w2_ref[...], preferred_element_type=jnp.float32) + b2_ref[...]
    )


def _edge_a(eap, w1bd, b1t, w2bd, b2t):
    return pl.pallas_call(
        _edge_a_kernel,
        grid=(E8 // BME8,),
        in_specs=[
            pl.BlockSpec((BME8, 128), lambda i: (i, 0)),
            pl.BlockSpec((128, 8 * D), lambda i: (0, 0)),
            pl.BlockSpec((1, 8 * D), lambda i: (0, 0)),
            pl.BlockSpec((128, 128), lambda i: (0, 0)),
            pl.BlockSpec((1, 128), lambda i: (0, 0)),
        ],
        out_specs=[
            pl.BlockSpec((BME8, 8 * D), lambda i: (i, 0)),
            pl.BlockSpec((BME8, 128), lambda i: (i, 0)),
        ],
        out_shape=[
            jax.ShapeDtypeStruct((E8, 8 * D), jnp.float32),
            jax.ShapeDtypeStruct((E8, 128), jnp.float32),
        ],
    )(eap, w1bd, b1t.reshape(1, -1), w2bd, b2t.reshape(1, -1))


def _edge_b_kernel(g1_ref, g2_ref, ae_ref, w_ref, b_ref, o_ref):
    ea2 = jnp.maximum(g1_ref[...] + g2_ref[...] + ae_ref[...], 0.0)
    o_ref[...] = (
        jnp.dot(ea2, w_ref[...], preferred_element_type=jnp.float32) + b_ref[...]
    )


def _edge_b(g1p, g2p, aep, wbd, bt):
    return pl.pallas_call(
        _edge_b_kernel,
        grid=(E8 // BME8,),
        in_specs=[
            pl.BlockSpec((BME8, 128), lambda i: (i, 0)),
            pl.BlockSpec((BME8, 128), lambda i: (i, 0)),
            pl.BlockSpec((BME8, 128), lambda i: (i, 0)),
            pl.BlockSpec((128, 8 * D), lambda i: (0, 0)),
            pl.BlockSpec((1, 8 * D), lambda i: (0, 0)),
        ],
        out_specs=pl.BlockSpec((BME8, 8 * D), lambda i: (i, 0)),
        out_shape=jax.ShapeDtypeStruct((E8, 8 * D), jnp.float32),
    )(g1p, g2p, aep, wbd, bt.reshape(1, -1))


def _mean_agg(sp_ref, cnt_ref, h_ref, wa_ref, wb_ref, ab_ref):
    s = sp_ref[0] + sp_ref[1]
    cnt = cnt_ref[0, :, 0:1] + cnt_ref[1, :, 0:1]
    inv = 1.0 / jnp.maximum(cnt, 1.0)
    aggr = s * inv
    out = jnp.maximum(
        jnp.dot(aggr, wa_ref[...], preferred_element_type=jnp.float32)
        + jnp.dot(h_ref[...], wb_ref[...], preferred_element_type=jnp.float32)
        + ab_ref[...],
        0.0,
    )
    nrm = jnp.maximum(jnp.sqrt(jnp.sum(out * out, axis=-1, keepdims=True)), 1e-12)
    return out / nrm


def _layer1_kernel(sp_ref, cnt_ref, x_ref, wa_ref, wb_ref, ab_ref,
                   we1_ref, we2_ref, wm_ref,
                   h_ref, he1_ref, he2_ref, hm_ref):
    h = _mean_agg(sp_ref, cnt_ref, x_ref, wa_ref, wb_ref, ab_ref)
    h_ref[...] = h
    he1_ref[...] = jnp.dot(h, we1_ref[...], preferred_element_type=jnp.float32)
    he2_ref[...] = jnp.dot(h, we2_ref[...], preferred_element_type=jnp.float32)
    hm_ref[...] = jnp.dot(h, wm_ref[...], preferred_element_type=jnp.float32)


def _layer1(sp, cnt, x, wa, wb, ab, we1, we2, wm):
    full = lambda r, c: pl.BlockSpec((r, c), lambda i: (0, 0))
    return pl.pallas_call(
        _layer1_kernel,
        grid=(N // BM_N,),
        in_specs=[
            pl.BlockSpec((NC, BM_N, D), lambda i: (0, i, 0)),
            pl.BlockSpec((NC, BM_N, D), lambda i: (0, i, 0)),
            pl.BlockSpec((BM_N, D), lambda i: (i, 0)),
            full(D, D), full(D, D), full(1, D),
            full(D, DE), full(D, DE), full(D, D),
        ],
        out_specs=[
            pl.BlockSpec((BM_N, D), lambda i: (i, 0)),
            pl.BlockSpec((BM_N, DE), lambda i: (i, 0)),
            pl.BlockSpec((BM_N, DE), lambda i: (i, 0)),
            pl.BlockSpec((BM_N, D), lambda i: (i, 0)),
        ],
        out_shape=[
            jax.ShapeDtypeStruct((N, D), jnp.float32),
            jax.ShapeDtypeStruct((N, DE), jnp.float32),
            jax.ShapeDtypeStruct((N, DE), jnp.float32),
            jax.ShapeDtypeStruct((N, D), jnp.float32),
        ],
    )(sp, cnt, x, wa, wb, ab.reshape(1, -1), we1, we2, wm)


def _layer2_kernel(sp_ref, cnt_ref, h_ref, wa_ref, wb_ref, ab_ref,
                   pw1_ref, pb1_ref, pw2_ref, pb2_ref, o_ref):
    h2 = _mean_agg(sp_ref, cnt_ref, h_ref, wa_ref, wb_ref, ab_ref)
    t = jnp.maximum(
        jnp.dot(h2, pw1_ref[...], preferred_element_type=jnp.float32) + pb1_ref[...],
        0.0,
    )
    o_ref[...] = (
        jnp.dot(t, pw2_ref[...], preferred_element_type=jnp.float32) + pb2_ref[...]
    )


def _layer2(sp, cnt, h, wa, wb, ab, pw1, pb1, pw2, pb2):
    full = lambda r, c: pl.BlockSpec((r, c), lambda i: (0, 0))
    return pl.pallas_call(
        _layer2_kernel,
        grid=(N // BM_N,),
        in_specs=[
            pl.BlockSpec((NC, BM_N, D), lambda i: (0, i, 0)),
            pl.BlockSpec((NC, BM_N, D), lambda i: (0, i, 0)),
            pl.BlockSpec((BM_N, D), lambda i: (i, 0)),
            full(D, D), full(D, D), full(1, D),
            full(D, D), full(1, D), full(D, D), full(1, D),
        ],
        out_specs=pl.BlockSpec((BM_N, D), lambda i: (i, 0)),
        out_shape=jax.ShapeDtypeStruct((N, D), jnp.float32),
    )(sp, cnt, h, wa, wb, ab.reshape(1, -1),
      pw1, pb1.reshape(1, -1), pw2, pb2.reshape(1, -1))


_NCH = E // CH
_NLOOP = -(-_NCH // NW)
_ROW_STRIDE = 624
_ROW_LEN = 640


def _seg_body(t_hbm, a_hbm, src_hbm, dst_hbm, sp_out,
              sr0, sr1, sr2, sr3, dr0, dr1, dr2, dr3,
              rw0, rw1, av0, av1, acc,
              i0, i1, i2, i3, g0, g1, a0, a1, s0, s1):
    c = lax.axis_index("c")
    s = lax.axis_index("s")
    wid = s * NC + c
    srcr = [sr0, sr1, sr2, sr3]
    dstr = [dr0, dr1, dr2, dr3]
    rows = [rw0, rw1]
    av = [av0, av1]
    isem = [i0, i1, i2, i3]
    gsem = [g0, g1]
    asem = [a0, a1]
    ssem = [s0, s1]
    zero16 = jnp.zeros((L,), jnp.float32)

    def _zr(i, _):
        for dd in range(D // L):
            rows[0][i, pl.ds(dd * L, L)] = zero16
        return 0
    lax.fori_loop(0, CH, _zr, 0)
    for kk in range(_ROW_LEN // CH):
        pltpu.sync_copy(
            rows[0],
            acc.at[pl.ds(s * _ROW_STRIDE + kk * CH, CH)],
        )
    plsc.subcore_barrier()

    def _idx_start(ck, slot):
        base = ck * CH
        pltpu.async_copy(src_hbm.at[pl.ds(base, CH)], srcr[slot], isem[slot])
        pltpu.async_copy(dst_hbm.at[pl.ds(base, CH)], dstr[slot], isem[slot])

    _idx_start(wid, 0)

    def _grp(g, _):
        for b in range(4):
            k = g * 4 + b
            b2 = b % 2
            b2m = (b - 1) % 2
            ck = k * NW + wid
            @pl.when(jnp.logical_and(k >= 2, ck - 2 * NW < _NCH))
            def _():
                pltpu.make_async_copy(
                    rows[b2], acc.at[dstr[(b + 2) % 4]], ssem[b2]
                ).wait()

            @pl.when(ck + NW < _NCH)
            def _():
                _idx_start(ck + NW, (b + 1) % 4)

            @pl.when(ck < _NCH)
            def _():
                pltpu.async_copy(
                    a_hbm.at[pl.ds(ck * (CH // 8), CH // 8)], av[b2], asem[b2]
                )
                pltpu.make_async_copy(
                    src_hbm.at[pl.ds(ck * CH, CH)], srcr[b], isem[b]
                ).wait()
                pltpu.make_async_copy(
                    dst_hbm.at[pl.ds(ck * CH, CH)], dstr[b], isem[b]
                ).wait()
                pltpu.async_copy(t_hbm.at[srcr[b]], rows[b2], gsem[b2])

            @pl.when(jnp.logical_and(k >= 1, ck - NW < _NCH))
            def _():
                pltpu.make_async_copy(
                    t_hbm.at[srcr[(b - 1) % 4]], rows[b2m], gsem[b2m]
                ).wait()
                pltpu.make_async_copy(
                    a_hbm.at[pl.ds((ck - NW) * (CH // 8), CH // 8)],
                    av[b2m], asem[b2m],
                ).wait()

                @plsc.parallel_loop(0, CH // 8, unroll=2)
                def _ew(pr):
                    for j in range(8):
                        ee = pr * 8 + j
                        for dd in range(D // L):
                            sl = pl.ds(dd * L, L)
                            rows[b2m][ee, sl] = jnp.maximum(
                                rows[b2m][ee, sl]
                                + av[b2m][pr, pl.ds(j * D + dd * L, L)],
                                0.0,
                            )

                pltpu.async_copy(
                    rows[b2m], acc.at[dstr[(b - 1) % 4]], ssem[b2m],
                    add=True,
                )
        return 0
    lax.fori_loop(0, (_NLOOP + 2 + 3) // 4, _grp, 0)
    plsc.subcore_barrier()

    pltpu.sync_copy(
        acc.at[pl.ds(s * _ROW_STRIDE, _ROW_LEN)],
        sp_out.at[c, pl.ds(s * _ROW_STRIDE, _ROW_LEN)],
    )


_seg_kernel = functools.partial(
    pl.kernel,
    out_type=jax.ShapeDtypeStruct((NC, N, D), jnp.float32),
    mesh=plsc.VectorSubcoreMesh(core_axis_name="c", subcore_axis_name="s", num_cores=NC, num_subcores=NS),
    scratch_types=[pltpu.VMEM((CH,), jnp.int32)] * 8 + [
        pltpu.VMEM((CH, D), jnp.float32),
        pltpu.VMEM((CH, D), jnp.float32),
        pltpu.VMEM((CH // 8, 8 * D), jnp.float32),
        pltpu.VMEM((CH // 8, 8 * D), jnp.float32),
        pltpu.VMEM_SHARED((N, D), jnp.float32),
    ] + [pltpu.SemaphoreType.DMA] * 10,
)(_seg_body)


def _count_body(dst_hbm, cnt_out, dr0, dr1, dr2, dr3, ones_v, acc,
                i0, i1, i2, i3, s0, s1):
    c = lax.axis_index("c")
    s = lax.axis_index("s")
    wid = s * NC + c
    dstr = [dr0, dr1, dr2, dr3]
    isem = [i0, i1, i2, i3]
    ssem = [s0, s1]
    zero16 = jnp.zeros((L,), jnp.float32)
    ones16 = jnp.ones((L,), jnp.float32)

    def _zr(i, _):
        for dd in range(D // L):
            ones_v[i, pl.ds(dd * L, L)] = zero16
        return 0
    lax.fori_loop(0, CH, _zr, 0)

    for kk in range(_ROW_LEN // CH):
        pltpu.sync_copy(
            ones_v,
            acc.at[pl.ds(s * _ROW_STRIDE + kk * CH, CH)],
        )

    def _o1(i, _):
        for dd in range(D // L):
            ones_v[i, pl.ds(dd * L, L)] = ones16
        return 0
    lax.fori_loop(0, CH, _o1, 0)
    plsc.subcore_barrier()

    pltpu.async_copy(dst_hbm.at[pl.ds(wid * CH, CH)], dstr[0], isem[0])

    def _grp(g, _):
        for b in range(4):
            k = g * 4 + b
            b2 = b % 2
            ck = k * NW + wid

            @pl.when(jnp.logical_and(k >= 2, ck - 2 * NW < _NCH))
            def _():
                pltpu.make_async_copy(
                    ones_v, acc.at[dstr[(b + 2) % 4]], ssem[b2]
                ).wait()

            @pl.when(ck + NW < _NCH)
            def _():
                pltpu.async_copy(
                    dst_hbm.at[pl.ds((ck + NW) * CH, CH)],
                    dstr[(b + 1) % 4], isem[(b + 1) % 4],
                )

            @pl.when(ck < _NCH)
            def _():
                pltpu.make_async_copy(
                    dst_hbm.at[pl.ds(ck * CH, CH)], dstr[b], isem[b]
                ).wait()
                pltpu.async_copy(
                    ones_v, acc.at[dstr[b]], ssem[b2], add=True
                )
        return 0
    lax.fori_loop(0, (_NLOOP + 2 + 3) // 4, _grp, 0)
    plsc.subcore_barrier()

    pltpu.sync_copy(
        acc.at[pl.ds(s * _ROW_STRIDE, _ROW_LEN)],
        cnt_out.at[c, pl.ds(s * _ROW_STRIDE, _ROW_LEN)],
    )


_count_kernel = functools.partial(
    pl.kernel,
    out_type=jax.ShapeDtypeStruct((NC, N, D), jnp.float32),
    mesh=plsc.VectorSubcoreMesh(core_axis_name="c", subcore_axis_name="s", num_cores=NC, num_subcores=NS),
    scratch_types=[pltpu.VMEM((CH,), jnp.int32)] * 4 + [
        pltpu.VMEM((CH, D), jnp.float32),
        pltpu.VMEM_SHARED((N, D), jnp.float32),
    ] + [pltpu.SemaphoreType.DMA] * 6,
)(_count_body)


def _gather2_body(t1_hbm, t2_hbm, src_hbm, dst_hbm, o1_out, o2_out,
                  sr0, sr1, sr2, sr3, dr0, dr1, dr2, dr3,
                  g1a, g1b, g2a, g2b,
                  i0, i1, i2, i3, g0, g1, o0, o1):
    c = lax.axis_index("c")
    s = lax.axis_index("s")
    wid = s * NC + c
    srcr = [sr0, sr1, sr2, sr3]
    dstr = [dr0, dr1, dr2, dr3]
    g1_v = [g1a, g1b]
    g2_v = [g2a, g2b]
    isem = [i0, i1, i2, i3]
    gsem = [g0, g1]
    osem = [o0, o1]

    pltpu.async_copy(src_hbm.at[pl.ds(wid * CH, CH)], srcr[0], isem[0])
    pltpu.async_copy(dst_hbm.at[pl.ds(wid * CH, CH)], dstr[0], isem[0])

    def _grp(g, _):
        for b in range(4):
            k = g * 4 + b
            b2 = b % 2
            b2m = (b - 1) % 2
            ck = k * NW + wid

            @pl.when(jnp.logical_and(k >= 2, ck - 2 * NW < _NCH))
            def _():
                base = (ck - 2 * NW) * CH
                pltpu.make_async_copy(
                    g1_v[b2], o1_out.at[pl.ds(base, CH)], osem[b2]
                ).wait()
                pltpu.make_async_copy(
                    g2_v[b2], o2_out.at[pl.ds(base, CH)], osem[b2]
                ).wait()

            @pl.when(ck + NW < _NCH)
            def _():
                base = (ck + NW) * CH
                pltpu.async_copy(
                    src_hbm.at[pl.ds(base, CH)], srcr[(b + 1) % 4],
                    isem[(b + 1) % 4],
                )
                pltpu.async_copy(
                    dst_hbm.at[pl.ds(base, CH)], dstr[(b + 1) % 4],
                    isem[(b + 1) % 4],
                )

            @pl.when(ck < _NCH)
            def _():
                base = ck * CH
                pltpu.make_async_copy(
                    src_hbm.at[pl.ds(base, CH)], srcr[b], isem[b]
                ).wait()
                pltpu.make_async_copy(
                    dst_hbm.at[pl.ds(base, CH)], dstr[b], isem[b]
                ).wait()
                pltpu.async_copy(t1_hbm.at[srcr[b]], g1_v[b2], gsem[b2])
                pltpu.async_copy(t2_hbm.at[dstr[b]], g2_v[b2], gsem[b2])

            @pl.when(jnp.logical_and(k >= 1, ck - NW < _NCH))
            def _():
                base = (ck - NW) * CH
                pltpu.make_async_copy(
                    t1_hbm.at[srcr[(b - 1) % 4]], g1_v[b2m], gsem[b2m]
                ).wait()
                pltpu.make_async_copy(
                    t2_hbm.at[dstr[(b - 1) % 4]], g2_v[b2m], gsem[b2m]
                ).wait()
                pltpu.async_copy(
                    g1_v[b2m], o1_out.at[pl.ds(base, CH)], osem[b2m]
                )
                pltpu.async_copy(
                    g2_v[b2m], o2_out.at[pl.ds(base, CH)], osem[b2m]
                )
        return 0
    lax.fori_loop(0, (_NLOOP + 2 + 3) // 4, _grp, 0)


_gather2_kernel = functools.partial(
    pl.kernel,
    out_type=[
        jax.ShapeDtypeStruct((E, DE), jnp.float32),
        jax.ShapeDtypeStruct((E, DE), jnp.float32),
    ],
    compiler_params=pltpu.CompilerParams(use_tc_tiling_on_sc=False),
    mesh=plsc.VectorSubcoreMesh(core_axis_name="c", subcore_axis_name="s", num_cores=NC, num_subcores=NS),
    scratch_types=[pltpu.VMEM((CH,), jnp.int32)] * 8 + [
        pltpu.VMEM((CH, DE), jnp.float32),
        pltpu.VMEM((CH, DE), jnp.float32),
        pltpu.VMEM((CH, DE), jnp.float32),
        pltpu.VMEM((CH, DE), jnp.float32),
    ] + [pltpu.SemaphoreType.DMA] * 8,
)(_gather2_body)


def kernel(x, edge_attr, edge_index, msg_W1, msg_b1, agg_W1, agg_b1,
           edge_W, edge_b, msg_W2, msg_b2, agg_W2, agg_b2,
           post_W1, post_b1, post_W2, post_b2):
    src = edge_index[0]
    dst = edge_index[1]
    eye8 = jnp.eye(8, dtype=jnp.float32)
    eap = edge_attr.reshape(E8, 128)

    xW = _mm(x, msg_W1[:D], jnp.zeros((D,), jnp.float32), BM_N)
    a1, ae = _edge_a(
        eap,
        jnp.kron(eye8, msg_W1[D:]), jnp.tile(msg_b1, 8),
        jnp.kron(eye8, edge_W[2 * D:]), jnp.tile(edge_b, 8),
    )

    cnt = _count_kernel(dst)
    sp1 = _seg_kernel(xW, a1, src, dst)

    h, he1, he2, hm = _layer1(
        sp1, cnt, x, agg_W1[:D], agg_W1[D:], agg_b1,
        edge_W[:D], edge_W[D:2 * D], msg_W2[:D],
    )

    g1, g2 = _gather2_kernel(he1, he2, src, dst)

    a2 = _edge_b(
        g1.reshape(E8, 128), g2.reshape(E8, 128), ae,
        jnp.kron(eye8, msg_W2[D:]), jnp.tile(msg_b2, 8),
    )

    sp2 = _seg_kernel(hm, a2, src, dst)

    return _layer2(
        sp2, cnt, h, agg_W2[:D], agg_W2[D:], agg_b2,
        post_W1, post_b1, post_W2, post_b2,
    )


# speedup vs baseline: 1.0246x; 1.0246x over previous
import functools

import jax
import jax.numpy as jnp
from jax import lax
from jax.experimental import pallas as pl
from jax.experimental.pallas import tpu as pltpu
from jax.experimental.pallas import tpu_sc as plsc

N = 10000
E = 320000
D = 128
DE = 16

NC = 2
NS = 16
L = 16
NW = NC * NS
CH = 64

BM_E = 8000
BM_N = 2000


def _mm_kernel(x_ref, w_ref, b_ref, o_ref):
    o_ref[...] = (
        jnp.dot(x_ref[...], w_ref[...], preferred_element_type=jnp.float32)
        + b_ref[...]
    )


def _mm(x, w, b, bm):
    m, k = x.shape
    dout = w.shape[1]
    return pl.pallas_call(
        _mm_kernel,
        grid=(m // bm,),
        in_specs=[
            pl.BlockSpec((bm, k), lambda i: (i, 0)),
            pl.BlockSpec((k, dout), lambda i: (0, 0)),
            pl.BlockSpec((1, dout), lambda i: (0, 0)),
        ],
        out_specs=pl.BlockSpec((bm, dout), lambda i: (i, 0)),
        out_shape=jax.ShapeDtypeStruct((m, dout), jnp.float32),
    )(x, w, b.reshape(1, -1))


E8 = E // 8
BME8 = BM_E // 8


def _edge_a_kernel(ea_ref, w1_ref, b1_ref, w2_ref, b2_ref, o1_ref, o2_ref):
    ea = ea_ref[...]
    o1_ref[...] = (
        jnp.dot(ea, w1_ref[...], preferred_element_type=jnp.float32) + b1_ref[...]
    )
    o2_ref[...] = (
        jnp.dot(ea, w2_ref[...], preferred_element_type=jnp.float32) + b2_ref[...]
    )


def _edge_a(eap, w1bd, b1t, w2bd, b2t):
    return pl.pallas_call(
        _edge_a_kernel,
        grid=(E8 // BME8,),
        in_specs=[
            pl.BlockSpec((BME8, 128), lambda i: (i, 0)),
            pl.BlockSpec((128, 8 * D), lambda i: (0, 0)),
            pl.BlockSpec((1, 8 * D), lambda i: (0, 0)),
            pl.BlockSpec((128, 128), lambda i: (0, 0)),
            pl.BlockSpec((1, 128), lambda i: (0, 0)),
        ],
        out_specs=[
            pl.BlockSpec((BME8, 8 * D), lambda i: (i, 0)),
            pl.BlockSpec((BME8, 128), lambda i: (i, 0)),
        ],
        out_shape=[
            jax.ShapeDtypeStruct((E8, 8 * D), jnp.float32),
            jax.ShapeDtypeStruct((E8, 128), jnp.float32),
        ],
    )(eap, w1bd, b1t.reshape(1, -1), w2bd, b2t.reshape(1, -1))


def _edge_b_kernel(g1_ref, g2_ref, ae_ref, w_ref, b_ref, o_ref):
    ea2 = jnp.maximum(g1_ref[...] + g2_ref[...] + ae_ref[...], 0.0)
    o_ref[...] = (
        jnp.dot(ea2, w_ref[...], preferred_element_type=jnp.float32) + b_ref[...]
    )


def _edge_b(g1p, g2p, aep, wbd, bt):
    return pl.pallas_call(
        _edge_b_kernel,
        grid=(E8 // BME8,),
        in_specs=[
            pl.BlockSpec((BME8, 128), lambda i: (i, 0)),
            pl.BlockSpec((BME8, 128), lambda i: (i, 0)),
            pl.BlockSpec((BME8, 128), lambda i: (i, 0)),
            pl.BlockSpec((128, 8 * D), lambda i: (0, 0)),
            pl.BlockSpec((1, 8 * D), lambda i: (0, 0)),
        ],
        out_specs=pl.BlockSpec((BME8, 8 * D), lambda i: (i, 0)),
        out_shape=jax.ShapeDtypeStruct((E8, 8 * D), jnp.float32),
    )(g1p, g2p, aep, wbd, bt.reshape(1, -1))


def _mean_agg(sp_ref, cnt_ref, h_ref, wa_ref, wb_ref, ab_ref):
    s = sp_ref[0] + sp_ref[1]
    cnt = cnt_ref[0, :, 0:1] + cnt_ref[1, :, 0:1]
    inv = 1.0 / jnp.maximum(cnt, 1.0)
    aggr = s * inv
    out = jnp.maximum(
        jnp.dot(aggr, wa_ref[...], preferred_element_type=jnp.float32)
        + jnp.dot(h_ref[...], wb_ref[...], preferred_element_type=jnp.float32)
        + ab_ref[...],
        0.0,
    )
    nrm = jnp.maximum(jnp.sqrt(jnp.sum(out * out, axis=-1, keepdims=True)), 1e-12)
    return out / nrm


def _layer1_kernel(sp_ref, cnt_ref, x_ref, wa_ref, wb_ref, ab_ref,
                   we1_ref, we2_ref, wm_ref,
                   h_ref, he1_ref, he2_ref, hm_ref):
    h = _mean_agg(sp_ref, cnt_ref, x_ref, wa_ref, wb_ref, ab_ref)
    h_ref[...] = h
    he1_ref[...] = jnp.dot(h, we1_ref[...], preferred_element_type=jnp.float32)
    he2_ref[...] = jnp.dot(h, we2_ref[...], preferred_element_type=jnp.float32)
    hm_ref[...] = jnp.dot(h, wm_ref[...], preferred_element_type=jnp.float32)


def _layer1(sp, cnt, x, wa, wb, ab, we1, we2, wm):
    full = lambda r, c: pl.BlockSpec((r, c), lambda i: (0, 0))
    return pl.pallas_call(
        _layer1_kernel,
        grid=(N // BM_N,),
        in_specs=[
            pl.BlockSpec((NC, BM_N, D), lambda i: (0, i, 0)),
            pl.BlockSpec((NC, BM_N, D), lambda i: (0, i, 0)),
            pl.BlockSpec((BM_N, D), lambda i: (i, 0)),
            full(D, D), full(D, D), full(1, D),
            full(D, DE), full(D, DE), full(D, D),
        ],
        out_specs=[
            pl.BlockSpec((BM_N, D), lambda i: (i, 0)),
            pl.BlockSpec((BM_N, DE), lambda i: (i, 0)),
            pl.BlockSpec((BM_N, DE), lambda i: (i, 0)),
            pl.BlockSpec((BM_N, D), lambda i: (i, 0)),
        ],
        out_shape=[
            jax.ShapeDtypeStruct((N, D), jnp.float32),
            jax.ShapeDtypeStruct((N, DE), jnp.float32),
            jax.ShapeDtypeStruct((N, DE), jnp.float32),
            jax.ShapeDtypeStruct((N, D), jnp.float32),
        ],
    )(sp, cnt, x, wa, wb, ab.reshape(1, -1), we1, we2, wm)


def _layer2_kernel(sp_ref, cnt_ref, h_ref, wa_ref, wb_ref, ab_ref,
                   pw1_ref, pb1_ref, pw2_ref, pb2_ref, o_ref):
    h2 = _mean_agg(sp_ref, cnt_ref, h_ref, wa_ref, wb_ref, ab_ref)
    t = jnp.maximum(
        jnp.dot(h2, pw1_ref[...], preferred_element_type=jnp.float32) + pb1_ref[...],
        0.0,
    )
    o_ref[...] = (
        jnp.dot(t, pw2_ref[...], preferred_element_type=jnp.float32) + pb2_ref[...]
    )


def _layer2(sp, cnt, h, wa, wb, ab, pw1, pb1, pw2, pb2):
    full = lambda r, c: pl.BlockSpec((r, c), lambda i: (0, 0))
    return pl.pallas_call(
        _layer2_kernel,
        grid=(N // BM_N,),
        in_specs=[
            pl.BlockSpec((NC, BM_N, D), lambda i: (0, i, 0)),
            pl.BlockSpec((NC, BM_N, D), lambda i: (0, i, 0)),
            pl.BlockSpec((BM_N, D), lambda i: (i, 0)),
            full(D, D), full(D, D), full(1, D),
            full(D, D), full(1, D), full(D, D), full(1, D),
        ],
        out_specs=pl.BlockSpec((BM_N, D), lambda i: (i, 0)),
        out_shape=jax.ShapeDtypeStruct((N, D), jnp.float32),
    )(sp, cnt, h, wa, wb, ab.reshape(1, -1),
      pw1, pb1.reshape(1, -1), pw2, pb2.reshape(1, -1))


_NCH = E // CH
_NLOOP = -(-_NCH // NW)
_ROW_STRIDE = 624
_ROW_LEN = 640


def _seg_body(t_hbm, a_hbm, src_hbm, dst_hbm, sp_out,
              sr0, sr1, sr2, sr3, dr0, dr1, dr2, dr3,
              rw0, rw1, av0, av1, acc,
              i0, i1, i2, i3, g0, g1, a0, a1, s0, s1):
    c = lax.axis_index("c")
    s = lax.axis_index("s")
    wid = s * NC + c
    srcr = [sr0, sr1, sr2, sr3]
    dstr = [dr0, dr1, dr2, dr3]
    rows = [rw0, rw1]
    av = [av0, av1]
    isem = [i0, i1, i2, i3]
    gsem = [g0, g1]
    asem = [a0, a1]
    ssem = [s0, s1]
    zero16 = jnp.zeros((L,), jnp.float32)

    def _zr(i, _):
        for dd in range(D // L):
            rows[0][i, pl.ds(dd * L, L)] = zero16
        return 0
    lax.fori_loop(0, CH, _zr, 0)
    for kk in range(_ROW_LEN // CH):
        pltpu.sync_copy(
            rows[0],
            acc.at[pl.ds(s * _ROW_STRIDE + kk * CH, CH)],
        )
    plsc.subcore_barrier()

    def _idx_start(ck, slot):
        base = ck * CH
        pltpu.async_copy(src_hbm.at[pl.ds(base, CH)], srcr[slot], isem[slot])
        pltpu.async_copy(dst_hbm.at[pl.ds(base, CH)], dstr[slot], isem[slot])

    _idx_start(wid, 0)

    def _grp(g, _):
        for b in range(4):
            k = g * 4 + b
            b2 = b % 2
            b2m = (b - 1) % 2
            ck = k * NW + wid
            @pl.when(jnp.logical_and(k >= 2, ck - 2 * NW < _NCH))
            def _():
                pltpu.make_async_copy(
                    rows[b2], acc.at[dstr[(b + 2) % 4]], ssem[b2]
                ).wait()

            @pl.when(ck + NW < _NCH)
            def _():
                _idx_start(ck + NW, (b + 1) % 4)

            @pl.when(ck < _NCH)
            def _():
                pltpu.async_copy(
                    a_hbm.at[pl.ds(ck * (CH // 8), CH // 8)], av[b2], asem[b2]
                )
                pltpu.make_async_copy(
                    src_hbm.at[pl.ds(ck * CH, CH)], srcr[b], isem[b]
                ).wait()
                pltpu.make_async_copy(
                    dst_hbm.at[pl.ds(ck * CH, CH)], dstr[b], isem[b]
                ).wait()
                pltpu.async_copy(t_hbm.at[srcr[b]], rows[b2], gsem[b2])

            @pl.when(jnp.logical_and(k >= 1, ck - NW < _NCH))
            def _():
                pltpu.make_async_copy(
                    t_hbm.at[srcr[(b - 1) % 4]], rows[b2m], gsem[b2m]
                ).wait()
                pltpu.make_async_copy(
                    a_hbm.at[pl.ds((ck - NW) * (CH // 8), CH // 8)],
                    av[b2m], asem[b2m],
                ).wait()

                @plsc.parallel_loop(0, CH // 8)
                def _ew(pr):
                    for j in range(8):
                        ee = pr * 8 + j
                        for dd in range(D // L):
                            sl = pl.ds(dd * L, L)
                            rows[b2m][ee, sl] = jnp.maximum(
                                rows[b2m][ee, sl]
                                + av[b2m][pr, pl.ds(j * D + dd * L, L)],
                                0.0,
                            )

                pltpu.async_copy(
                    rows[b2m], acc.at[dstr[(b - 1) % 4]], ssem[b2m],
                    add=True,
                )
        return 0
    lax.fori_loop(0, (_NLOOP + 2 + 3) // 4, _grp, 0)
    plsc.subcore_barrier()

    pltpu.sync_copy(
        acc.at[pl.ds(s * _ROW_STRIDE, _ROW_LEN)],
        sp_out.at[c, pl.ds(s * _ROW_STRIDE, _ROW_LEN)],
    )


_seg_kernel = functools.partial(
    pl.kernel,
    out_type=jax.ShapeDtypeStruct((NC, N, D), jnp.float32),
    mesh=plsc.VectorSubcoreMesh(core_axis_name="c", subcore_axis_name="s", num_cores=NC, num_subcores=NS),
    scratch_types=[pltpu.VMEM((CH,), jnp.int32)] * 8 + [
        pltpu.VMEM((CH, D), jnp.float32),
        pltpu.VMEM((CH, D), jnp.float32),
        pltpu.VMEM((CH // 8, 8 * D), jnp.float32),
        pltpu.VMEM((CH // 8, 8 * D), jnp.float32),
        pltpu.VMEM_SHARED((N, D), jnp.float32),
    ] + [pltpu.SemaphoreType.DMA] * 10,
)(_seg_body)


def _count_body(dst_hbm, cnt_out, dr0, dr1, dr2, dr3, ones_v, acc,
                i0, i1, i2, i3, s0, s1):
    c = lax.axis_index("c")
    s = lax.axis_index("s")
    wid = s * NC + c
    dstr = [dr0, dr1, dr2, dr3]
    isem = [i0, i1, i2, i3]
    ssem = [s0, s1]
    zero16 = jnp.zeros((L,), jnp.float32)
    ones16 = jnp.ones((L,), jnp.float32)

    def _zr(i, _):
        for dd in range(D // L):
            ones_v[i, pl.ds(dd * L, L)] = zero16
        return 0
    lax.fori_loop(0, CH, _zr, 0)

    for kk in range(_ROW_LEN // CH):
        pltpu.sync_copy(
            ones_v,
            acc.at[pl.ds(s * _ROW_STRIDE + kk * CH, CH)],
        )

    def _o1(i, _):
        for dd in range(D // L):
            ones_v[i, pl.ds(dd * L, L)] = ones16
        return 0
    lax.fori_loop(0, CH, _o1, 0)
    plsc.subcore_barrier()

    pltpu.async_copy(dst_hbm.at[pl.ds(wid * CH, CH)], dstr[0], isem[0])

    def _grp(g, _):
        for b in range(4):
            k = g * 4 + b
            b2 = b % 2
            ck = k * NW + wid

            @pl.when(jnp.logical_and(k >= 2, ck - 2 * NW < _NCH))
            def _():
                pltpu.make_async_copy(
                    ones_v, acc.at[dstr[(b + 2) % 4]], ssem[b2]
                ).wait()

            @pl.when(ck + NW < _NCH)
            def _():
                pltpu.async_copy(
                    dst_hbm.at[pl.ds((ck + NW) * CH, CH)],
                    dstr[(b + 1) % 4], isem[(b + 1) % 4],
                )

            @pl.when(ck < _NCH)
            def _():
                pltpu.make_async_copy(
                    dst_hbm.at[pl.ds(ck * CH, CH)], dstr[b], isem[b]
                ).wait()
                pltpu.async_copy(
                    ones_v, acc.at[dstr[b]], ssem[b2], add=True
                )
        return 0
    lax.fori_loop(0, (_NLOOP + 2 + 3) // 4, _grp, 0)
    plsc.subcore_barrier()

    pltpu.sync_copy(
        acc.at[pl.ds(s * _ROW_STRIDE, _ROW_LEN)],
        cnt_out.at[c, pl.ds(s * _ROW_STRIDE, _ROW_LEN)],
    )


_count_kernel = functools.partial(
    pl.kernel,
    out_type=jax.ShapeDtypeStruct((NC, N, D), jnp.float32),
    mesh=plsc.VectorSubcoreMesh(core_axis_name="c", subcore_axis_name="s", num_cores=NC, num_subcores=NS),
    scratch_types=[pltpu.VMEM((CH,), jnp.int32)] * 4 + [
        pltpu.VMEM((CH, D), jnp.float32),
        pltpu.VMEM_SHARED((N, D), jnp.float32),
    ] + [pltpu.SemaphoreType.DMA] * 6,
)(_count_body)


def _gather2_body(t1_hbm, t2_hbm, src_hbm, dst_hbm, o1_out, o2_out,
                  sr0, sr1, sr2, sr3, dr0, dr1, dr2, dr3,
                  g1a, g1b, g2a, g2b,
                  i0, i1, i2, i3, g0, g1, o0, o1):
    c = lax.axis_index("c")
    s = lax.axis_index("s")
    wid = s * NC + c
    srcr = [sr0, sr1, sr2, sr3]
    dstr = [dr0, dr1, dr2, dr3]
    g1_v = [g1a, g1b]
    g2_v = [g2a, g2b]
    isem = [i0, i1, i2, i3]
    gsem = [g0, g1]
    osem = [o0, o1]

    pltpu.async_copy(src_hbm.at[pl.ds(wid * CH, CH)], srcr[0], isem[0])
    pltpu.async_copy(dst_hbm.at[pl.ds(wid * CH, CH)], dstr[0], isem[0])

    def _grp(g, _):
        for b in range(4):
            k = g * 4 + b
            b2 = b % 2
            b2m = (b - 1) % 2
            ck = k * NW + wid

            @pl.when(jnp.logical_and(k >= 2, ck - 2 * NW < _NCH))
            def _():
                base = (ck - 2 * NW) * CH
                pltpu.make_async_copy(
                    g1_v[b2], o1_out.at[pl.ds(base, CH)], osem[b2]
                ).wait()
                pltpu.make_async_copy(
                    g2_v[b2], o2_out.at[pl.ds(base, CH)], osem[b2]
                ).wait()

            @pl.when(ck + NW < _NCH)
            def _():
                base = (ck + NW) * CH
                pltpu.async_copy(
                    src_hbm.at[pl.ds(base, CH)], srcr[(b + 1) % 4],
                    isem[(b + 1) % 4],
                )
                pltpu.async_copy(
                    dst_hbm.at[pl.ds(base, CH)], dstr[(b + 1) % 4],
                    isem[(b + 1) % 4],
                )

            @pl.when(ck < _NCH)
            def _():
                base = ck * CH
                pltpu.make_async_copy(
                    src_hbm.at[pl.ds(base, CH)], srcr[b], isem[b]
                ).wait()
                pltpu.make_async_copy(
                    dst_hbm.at[pl.ds(base, CH)], dstr[b], isem[b]
                ).wait()
                pltpu.async_copy(t1_hbm.at[srcr[b]], g1_v[b2], gsem[b2])
                pltpu.async_copy(t2_hbm.at[dstr[b]], g2_v[b2], gsem[b2])

            @pl.when(jnp.logical_and(k >= 1, ck - NW < _NCH))
            def _():
                base = (ck - NW) * CH
                pltpu.make_async_copy(
                    t1_hbm.at[srcr[(b - 1) % 4]], g1_v[b2m], gsem[b2m]
                ).wait()
                pltpu.make_async_copy(
                    t2_hbm.at[dstr[(b - 1) % 4]], g2_v[b2m], gsem[b2m]
                ).wait()
                pltpu.async_copy(
                    g1_v[b2m], o1_out.at[pl.ds(base, CH)], osem[b2m]
                )
                pltpu.async_copy(
                    g2_v[b2m], o2_out.at[pl.ds(base, CH)], osem[b2m]
                )
        return 0
    lax.fori_loop(0, (_NLOOP + 2 + 3) // 4, _grp, 0)


_gather2_kernel = functools.partial(
    pl.kernel,
    out_type=[
        jax.ShapeDtypeStruct((E, DE), jnp.float32),
        jax.ShapeDtypeStruct((E, DE), jnp.float32),
    ],
    compiler_params=pltpu.CompilerParams(use_tc_tiling_on_sc=False),
    mesh=plsc.VectorSubcoreMesh(core_axis_name="c", subcore_axis_name="s", num_cores=NC, num_subcores=NS),
    scratch_types=[pltpu.VMEM((CH,), jnp.int32)] * 8 + [
        pltpu.VMEM((CH, DE), jnp.float32),
        pltpu.VMEM((CH, DE), jnp.float32),
        pltpu.VMEM((CH, DE), jnp.float32),
        pltpu.VMEM((CH, DE), jnp.float32),
    ] + [pltpu.SemaphoreType.DMA] * 8,
)(_gather2_body)


def kernel(x, edge_attr, edge_index, msg_W1, msg_b1, agg_W1, agg_b1,
           edge_W, edge_b, msg_W2, msg_b2, agg_W2, agg_b2,
           post_W1, post_b1, post_W2, post_b2):
    src = edge_index[0]
    dst = edge_index[1]
    eye8 = jnp.eye(8, dtype=jnp.float32)
    eap = edge_attr.reshape(E8, 128)

    xW = _mm(x, msg_W1[:D], jnp.zeros((D,), jnp.float32), BM_N)
    a1, ae = _edge_a(
        eap,
        jnp.kron(eye8, msg_W1[D:]), jnp.tile(msg_b1, 8),
        jnp.kron(eye8, edge_W[2 * D:]), jnp.tile(edge_b, 8),
    )

    cnt = _count_kernel(dst)
    sp1 = _seg_kernel(xW, a1, src, dst)

    h, he1, he2, hm = _layer1(
        sp1, cnt, x, agg_W1[:D], agg_W1[D:], agg_b1,
        edge_W[:D], edge_W[D:2 * D], msg_W2[:D],
    )

    g1, g2 = _gather2_kernel(he1, he2, src, dst)

    a2 = _edge_b(
        g1.reshape(E8, 128), g2.reshape(E8, 128), ae,
        jnp.kron(eye8, msg_W2[D:]), jnp.tile(msg_b2, 8),
    )

    sp2 = _seg_kernel(hm, a2, src, dst)

    return _layer2(
        sp2, cnt, h, agg_W2[:D], agg_W2[D:], agg_b2,
        post_W1, post_b1, post_W2, post_b2,
    )
